# baseline (speedup 1.0000x reference)
import functools

import jax
import jax.numpy as jnp
from jax import lax
from jax.experimental import pallas as pl
from jax.experimental.pallas import tpu as pltpu
from jax.experimental.pallas import tpu_sc as plsc

S = 8192
N = 4
D = 768
NC = 2
NS = 16
NW = NC * NS
ROWS_PER_W = S // NW
RB = 32
NBUF = 4
NBLK = ROWS_PER_W // RB

_mesh = plsc.VectorSubcoreMesh(core_axis_name="c", subcore_axis_name="s")


@functools.partial(
    pl.kernel,
    mesh=_mesh,
    out_type=jax.ShapeDtypeStruct((S, N, D), jnp.float32),
    scratch_types=(
        [pltpu.VMEM((RB, D), jnp.float32)] * NBUF
        + [pltpu.SemaphoreType.DMA] * (2 * NBUF)
    ),
)
def _sc_bcast(table_hbm, out_hbm, *refs):
    bufs = refs[:NBUF]
    srs = refs[NBUF:2 * NBUF]
    sws = refs[2 * NBUF:]
    wid = lax.axis_index("s") * NC + lax.axis_index("c")
    base0 = wid * ROWS_PER_W

    def blk(i):
        return pl.ds(base0 + i * RB, RB)

    reads = [
        pltpu.async_copy(table_hbm.at[blk(j)], bufs[j], srs[j])
        for j in range(NBUF)
    ]
    pend = [None] * NBUF
    for i in range(NBLK):
        b = i % NBUF
        reads[b].wait()
        pend[b] = [
            pltpu.async_copy(bufs[b], out_hbm.at[blk(i), n], sws[b])
            for n in range(N)
        ]
        if i + NBUF < NBLK:
            for w in pend[b]:
                w.wait()
            pend[b] = None
            reads[b] = pltpu.async_copy(table_hbm.at[blk(i + NBUF)], bufs[b], srs[b])
    for ws in pend:
        if ws is not None:
            for w in ws:
                w.wait()


def kernel(x, pos_embedding):
    del x
    return _sc_bcast(pos_embedding)


# speedup vs baseline: 1.0671x; 1.0671x over previous
import functools

import jax
import jax.numpy as jnp
from jax import lax
from jax.experimental import pallas as pl
from jax.experimental.pallas import tpu as pltpu
from jax.experimental.pallas import tpu_sc as plsc

S = 8192
N = 4
D = 768
NC = 2
NS = 16
NW = NC * NS
ROWS_PER_W = S // NW
RB = 64
NBLK = ROWS_PER_W // RB

_mesh = plsc.VectorSubcoreMesh(core_axis_name="c", subcore_axis_name="s")


@functools.partial(
    pl.kernel,
    mesh=_mesh,
    out_type=jax.ShapeDtypeStruct((S, N, D), jnp.float32),
    scratch_types=[
        pltpu.VMEM((RB, D), jnp.float32),
        pltpu.VMEM((RB, D), jnp.float32),
        pltpu.SemaphoreType.DMA,
        pltpu.SemaphoreType.DMA,
    ],
)
def _sc_bcast(table_hbm, out_hbm, buf0, buf1, sr, sw):
    wid = lax.axis_index("s") * NC + lax.axis_index("c")
    base0 = wid * ROWS_PER_W
    bufs = (buf0, buf1)

    def blk(i):
        return pl.ds(base0 + i * RB, RB)

    reads = [
        pltpu.async_copy(table_hbm.at[blk(0)], buf0, sr),
        pltpu.async_copy(table_hbm.at[blk(1)], buf1, sr),
    ]
    tail_writes = []
    for i in range(NBLK):
        b = i % 2
        reads[b].wait()
        ws = [
            pltpu.async_copy(bufs[b], out_hbm.at[blk(i), n], sw)
            for n in range(N)
        ]
        if i + 2 < NBLK:
            for w in ws:
                w.wait()
            reads[b] = pltpu.async_copy(table_hbm.at[blk(i + 2)], bufs[b], sr)
        else:
            tail_writes.extend(ws)
    for w in tail_writes:
        w.wait()


def kernel(x, pos_embedding):
    del x
    return _sc_bcast(pos_embedding)
